# Initial kernel scaffold; baseline (speedup 1.0000x reference)
import functools

import jax
import jax.numpy as jnp
from jax import lax
from jax.experimental import pallas as pl
from jax.experimental.pallas import tpu as pltpu
from jax.experimental.pallas import tpu_sc as plsc

LANES = 16
NUM_CORES = 2
NUM_SUBCORES = 16
NW = NUM_CORES * NUM_SUBCORES
CHUNK = 128


def _sc_spmm(num_op, n, d, k_chunks):
    rows_per_tile = n // NUM_SUBCORES
    zcopy = 125
    n_zcopy = rows_per_tile // zcopy
    qs = d // LANES

    mesh = plsc.VectorSubcoreMesh(core_axis_name="c", subcore_axis_name="s")

    @functools.partial(
        pl.kernel,
        mesh=mesh,
        out_type=jax.ShapeDtypeStruct((NUM_CORES, n, d), jnp.float32),
        scratch_types=[
            pltpu.VMEM((LANES,), jnp.float32),
            pltpu.VMEM((k_chunks, CHUNK), jnp.int32),
            pltpu.VMEM((k_chunks, CHUNK), jnp.int32),
            pltpu.VMEM((k_chunks, CHUNK), jnp.float32),
            pltpu.VMEM((CHUNK, d), jnp.float32),
            pltpu.VMEM_SHARED((n, d), jnp.float32),
            pltpu.SemaphoreType.DMA,
        ],
    )
    def k(x_hbm, src_hbm, dst_hbm, val_hbm, ws_hbm, out_hbm,
          ws_v, src_v, dst_v, val_v, rows_v, acc, sem):
        c = lax.axis_index("c")
        s = lax.axis_index("s")
        wid = s * NUM_CORES + c

        def _zrow(r, carry):
            for q in range(qs):
                rows_v[r, pl.ds(q * LANES, LANES)] = jnp.zeros((LANES,), jnp.float32)
            return carry
        lax.fori_loop(0, CHUNK, _zrow, 0)

        base = s * rows_per_tile
        for z in range(n_zcopy):
            pltpu.sync_copy(rows_v.at[pl.ds(0, zcopy)],
                            acc.at[pl.ds(base + z * zcopy, zcopy)])
        plsc.subcore_barrier()

        pltpu.sync_copy(ws_hbm, ws_v)

        for i in range(num_op):
            pltpu.sync_copy(src_hbm.at[i, wid], src_v)
            pltpu.sync_copy(dst_hbm.at[i, wid], dst_v)
            pltpu.sync_copy(val_hbm.at[i, wid], val_v)
            wfac = ws_v[i]

            def _vscale(jj, carry):
                for q in range(CHUNK // LANES):
                    sl = pl.ds(q * LANES, LANES)
                    val_v[jj, sl] = val_v[jj, sl] * wfac
                return carry
            lax.fori_loop(0, k_chunks, _vscale, 0)

            def _chunk(j, carry):
                pltpu.async_copy(x_hbm.at[src_v.at[j]], rows_v, sem).wait()

                def _scale_row(r, cc):
                    sval = val_v[j, r]
                    for q in range(qs):
                        sl = pl.ds(q * LANES, LANES)
                        rows_v[r, sl] = rows_v[r, sl] * sval
                    return cc
                lax.fori_loop(0, CHUNK, _scale_row, 0)

                pltpu.sync_copy(rows_v, acc.at[dst_v.at[j]], add=True)
                return carry
            lax.fori_loop(0, k_chunks, _chunk, 0)

        plsc.subcore_barrier()
        for z in range(n_zcopy):
            sl = pl.ds(base + z * zcopy, zcopy)
            pltpu.sync_copy(acc.at[sl], out_hbm.at[c].at[sl])

    return k


def _combine(p_ref, o_ref):
    o_ref[...] = p_ref[0] + p_ref[1]


def kernel(x, adj_indices, adj_values, ws):
    n, d = x.shape
    num_op, _, e = adj_indices.shape
    k_chunks = -(-e // (NW * CHUNK))
    e_pad = NW * k_chunks * CHUNK
    pad = e_pad - e

    src = jnp.pad(adj_indices[:, 1, :], ((0, 0), (0, pad))).reshape(
        num_op, NW, k_chunks, CHUNK)
    dst = jnp.pad(adj_indices[:, 0, :], ((0, 0), (0, pad))).reshape(
        num_op, NW, k_chunks, CHUNK)
    val = jnp.pad(adj_values, ((0, 0), (0, pad))).reshape(
        num_op, NW, k_chunks, CHUNK)
    wsp = jnp.pad(ws / jnp.float32(num_op), (0, LANES - num_op))

    partials = _sc_spmm(num_op, n, d, k_chunks)(x, src, dst, val, wsp)

    blk = 1000
    return pl.pallas_call(
        _combine,
        grid=(n // blk,),
        in_specs=[pl.BlockSpec((NUM_CORES, blk, d), lambda i: (0, i, 0))],
        out_specs=pl.BlockSpec((blk, d), lambda i: (i, 0)),
        out_shape=jax.ShapeDtypeStruct((n, d), jnp.float32),
    )(partials)


# speedup vs baseline: 2.8942x; 2.8942x over previous
import functools

import jax
import jax.numpy as jnp
from jax import lax
from jax.experimental import pallas as pl
from jax.experimental.pallas import tpu as pltpu
from jax.experimental.pallas import tpu_sc as plsc

LANES = 16
NUM_CORES = 2
NUM_SUBCORES = 16
NW = NUM_CORES * NUM_SUBCORES
CHUNK = 128


def _sc_spmm(num_op, n, d, k_chunks):
    zcopy = CHUNK
    n_zcopy = -(-n // (NUM_SUBCORES * zcopy))
    rows_per_tile = n_zcopy * zcopy
    n_acc = NUM_SUBCORES * rows_per_tile
    qs = d // LANES

    mesh = plsc.VectorSubcoreMesh(core_axis_name="c", subcore_axis_name="s")

    @functools.partial(
        pl.kernel,
        mesh=mesh,
        compiler_params=pltpu.CompilerParams(needs_layout_passes=False),
        out_type=jax.ShapeDtypeStruct((NUM_CORES, n_acc, d), jnp.float32),
        scratch_types=[
            pltpu.VMEM((num_op, LANES), jnp.float32),
            pltpu.VMEM((k_chunks, CHUNK), jnp.int32),
            pltpu.VMEM((k_chunks, CHUNK), jnp.int32),
            pltpu.VMEM((k_chunks, CHUNK), jnp.float32),
            pltpu.VMEM((CHUNK, d), jnp.float32),
            pltpu.VMEM_SHARED((n_acc, d), jnp.float32),
            pltpu.SemaphoreType.DMA,
        ],
    )
    def k(x_hbm, src_hbm, dst_hbm, val_hbm, ws_hbm, out_hbm,
          ws_v, src_v, dst_v, val_v, rows_v, acc, sem):
        c = lax.axis_index("c")
        s = lax.axis_index("s")
        wid = s * NUM_CORES + c

        def _zrow(r, carry):
            for q in range(qs):
                rows_v[r, pl.ds(q * LANES, LANES)] = jnp.zeros((LANES,), jnp.float32)
            return carry
        lax.fori_loop(0, CHUNK, _zrow, 0)

        base = s * rows_per_tile
        for z in range(n_zcopy):
            pltpu.sync_copy(rows_v.at[pl.ds(0, zcopy)],
                            acc.at[pl.ds(base + z * zcopy, zcopy)])
        plsc.subcore_barrier()

        pltpu.sync_copy(ws_hbm, ws_v)

        for i in range(num_op):
            pltpu.sync_copy(src_hbm.at[i, wid], src_v)
            pltpu.sync_copy(dst_hbm.at[i, wid], dst_v)
            pltpu.sync_copy(val_hbm.at[i, wid], val_v)
            wvec = ws_v[i]

            def _vscale(jj, carry):
                for q in range(CHUNK // LANES):
                    sl = pl.ds(q * LANES, LANES)
                    val_v[jj, sl] = val_v[jj, sl] * wvec
                return carry
            lax.fori_loop(0, k_chunks, _vscale, 0)

            def _chunk(j, carry):
                pltpu.async_copy(x_hbm.at[src_v.at[j]], rows_v, sem).wait()

                jv = jnp.full((LANES,), j, dtype=jnp.int32)

                def _scale_row(r, cc):
                    sval = plsc.load_gather(
                        val_v, [jv, jnp.full((LANES,), r, dtype=jnp.int32)])
                    for q in range(qs):
                        sl = pl.ds(q * LANES, LANES)
                        rows_v[r, sl] = rows_v[r, sl] * sval
                    return cc
                lax.fori_loop(0, CHUNK, _scale_row, 0)

                pltpu.sync_copy(rows_v, acc.at[dst_v.at[j]], add=True)
                return carry
            lax.fori_loop(0, k_chunks, _chunk, 0)

        plsc.subcore_barrier()
        for z in range(n_zcopy):
            sl = pl.ds(base + z * zcopy, zcopy)
            pltpu.sync_copy(acc.at[sl], out_hbm.at[c].at[sl])

    return k


def _combine(p_ref, o_ref):
    o_ref[...] = p_ref[0] + p_ref[1]


def kernel(x, adj_indices, adj_values, ws):
    n, d = x.shape
    num_op, _, e = adj_indices.shape
    k_chunks = -(-e // (NW * CHUNK))
    e_pad = NW * k_chunks * CHUNK
    pad = e_pad - e

    src = jnp.pad(adj_indices[:, 1, :], ((0, 0), (0, pad))).reshape(
        num_op, NW, k_chunks, CHUNK)
    dst = jnp.pad(adj_indices[:, 0, :], ((0, 0), (0, pad))).reshape(
        num_op, NW, k_chunks, CHUNK)
    val = jnp.pad(adj_values, ((0, 0), (0, pad))).reshape(
        num_op, NW, k_chunks, CHUNK)
    wsp = jnp.tile((ws / jnp.float32(num_op))[:, None], (1, LANES))

    partials = _sc_spmm(num_op, n, d, k_chunks)(x, src, dst, val, wsp)

    blk = 1000
    return pl.pallas_call(
        _combine,
        grid=(n // blk,),
        in_specs=[pl.BlockSpec((NUM_CORES, blk, d), lambda i: (0, i, 0))],
        out_specs=pl.BlockSpec((blk, d), lambda i: (i, 0)),
        out_shape=jax.ShapeDtypeStruct((n, d), jnp.float32),
    )(partials)
